# Initial kernel scaffold; baseline (speedup 1.0000x reference)
import jax
import jax.numpy as jnp
from jax.experimental import pallas as pl


def kernel(x, W1, W2):
    raise NotImplementedError("write your pallas kernel here")


# speedup vs baseline: 7.3827x; 7.3827x over previous
import jax
import jax.numpy as jnp
from jax.experimental import pallas as pl

DIM = 512
N_GROUPS = 8
N_CODES = 1024
SUB = DIM // N_GROUPS
T_TILE = 512


def _rvq_kernel(x_ref, w1_ref, w2_ref, qout_ref, idx_ref, loss_ref):
    b = pl.program_id(0)
    i = pl.program_id(1)

    @pl.when((b == 0) & (i == 0))
    def _init():
        loss_ref[...] = jnp.zeros_like(loss_ref)

    tt = x_ref.shape[2]
    step_sum = jnp.float32(0.0)
    for g in range(N_GROUPS):
        xg = x_ref[0, g * SUB:(g + 1) * SUB, :]
        r = xg
        zq_acc = None
        for s, w_ref in enumerate((w1_ref, w2_ref)):
            wg = w_ref[g]
            wsq = jnp.sum(wg * wg, axis=1, keepdims=True)
            rsq = jnp.sum(r * r, axis=0, keepdims=True)
            prod = jax.lax.dot_general(
                wg, r, (((1,), (0,)), ((), ())),
                precision=jax.lax.Precision.DEFAULT,
                preferred_element_type=jnp.float32)
            d = rsq + wsq - 2.0 * prod
            iota_k = jax.lax.broadcasted_iota(jnp.int32, (N_CODES, tt), 0)
            dmin = jnp.min(d, axis=0, keepdims=True)
            idx = jnp.min(jnp.where(d == dmin, iota_k, N_CODES),
                          axis=0)
            oh = (iota_k == idx[None, :]).astype(jnp.float32)
            zq = jax.lax.dot_general(
                wg, oh, (((0,), (0,)), ((), ())),
                preferred_element_type=jnp.float32)
            t = zq - r
            step_sum = step_sum + jnp.sum(t * t)
            zq_st = r + t
            r = r - zq_st
            zq_acc = zq_st if zq_acc is None else zq_acc + zq_st
            idx_ref[s, g, :] = idx
        qout_ref[0, g * SUB:(g + 1) * SUB, :] = zq_acc

    loss_ref[...] = loss_ref[...] + step_sum


def kernel(x, W1, W2):
    B, C, T = x.shape
    nt = T // T_TILE
    grid = (B, nt)
    qout, idx, loss = pl.pallas_call(
        _rvq_kernel,
        grid=grid,
        in_specs=[
            pl.BlockSpec((1, DIM, T_TILE), lambda b, i: (b, 0, i)),
            pl.BlockSpec((N_GROUPS, N_CODES, SUB), lambda b, i: (0, 0, 0)),
            pl.BlockSpec((N_GROUPS, N_CODES, SUB), lambda b, i: (0, 0, 0)),
        ],
        out_specs=[
            pl.BlockSpec((1, DIM, T_TILE), lambda b, i: (b, 0, i)),
            pl.BlockSpec((2, N_GROUPS, T_TILE),
                         lambda b, i, _nt=nt: (0, 0, b * _nt + i)),
            pl.BlockSpec((1, 1), lambda b, i: (0, 0)),
        ],
        out_shape=[
            jax.ShapeDtypeStruct((B, DIM, T), jnp.float32),
            jax.ShapeDtypeStruct((2, N_GROUPS, B * T), jnp.int32),
            jax.ShapeDtypeStruct((1, 1), jnp.float32),
        ],
    )(x, W1, W2)
    numel = B * C * T
    total_loss = loss[0, 0] * (1.25 / (2.0 * numel))
    return (qout, total_loss, idx)


# speedup vs baseline: 8.6315x; 1.1692x over previous
import jax
import jax.numpy as jnp
from jax.experimental import pallas as pl

DIM = 512
N_GROUPS = 8
N_CODES = 1024
SUB = DIM // N_GROUPS
T_TILE = 1024


def _rvq_kernel(x_ref, w1_ref, w2_ref, qout_ref, idx_ref, loss_ref):
    b = pl.program_id(0)
    i = pl.program_id(1)

    @pl.when((b == 0) & (i == 0))
    def _init():
        loss_ref[...] = jnp.zeros_like(loss_ref)

    tt = x_ref.shape[2]
    step_sum = jnp.float32(0.0)
    for g in range(N_GROUPS):
        xg = x_ref[0, g * SUB:(g + 1) * SUB, :]
        r = xg
        zq_acc = None
        for s, w_ref in enumerate((w1_ref, w2_ref)):
            wg = w_ref[g]
            wsq = jnp.sum(wg * wg, axis=1, keepdims=True)
            rsq = jnp.sum(r * r, axis=0, keepdims=True)
            prod = jax.lax.dot_general(
                wg, r, (((1,), (0,)), ((), ())),
                precision=jax.lax.Precision.DEFAULT,
                preferred_element_type=jnp.float32)
            d = rsq + wsq - 2.0 * prod
            iota_k = jax.lax.broadcasted_iota(jnp.int32, (N_CODES, tt), 0)
            dmin = jnp.min(d, axis=0, keepdims=True)
            idx = jnp.min(jnp.where(d == dmin, iota_k, N_CODES),
                          axis=0)
            oh = (iota_k == idx[None, :]).astype(jnp.float32)
            zq = jax.lax.dot_general(
                wg, oh, (((0,), (0,)), ((), ())),
                preferred_element_type=jnp.float32)
            t = zq - r
            step_sum = step_sum + jnp.sum(t * t)
            zq_st = r + t
            r = r - zq_st
            zq_acc = zq_st if zq_acc is None else zq_acc + zq_st
            idx_ref[s, g, :] = idx
        qout_ref[0, g * SUB:(g + 1) * SUB, :] = zq_acc

    loss_ref[...] = loss_ref[...] + step_sum


def kernel(x, W1, W2):
    B, C, T = x.shape
    nt = T // T_TILE
    grid = (B, nt)
    qout, idx, loss = pl.pallas_call(
        _rvq_kernel,
        grid=grid,
        in_specs=[
            pl.BlockSpec((1, DIM, T_TILE), lambda b, i: (b, 0, i)),
            pl.BlockSpec((N_GROUPS, N_CODES, SUB), lambda b, i: (0, 0, 0)),
            pl.BlockSpec((N_GROUPS, N_CODES, SUB), lambda b, i: (0, 0, 0)),
        ],
        out_specs=[
            pl.BlockSpec((1, DIM, T_TILE), lambda b, i: (b, 0, i)),
            pl.BlockSpec((2, N_GROUPS, T_TILE),
                         lambda b, i, _nt=nt: (0, 0, b * _nt + i)),
            pl.BlockSpec((1, 1), lambda b, i: (0, 0)),
        ],
        out_shape=[
            jax.ShapeDtypeStruct((B, DIM, T), jnp.float32),
            jax.ShapeDtypeStruct((2, N_GROUPS, B * T), jnp.int32),
            jax.ShapeDtypeStruct((1, 1), jnp.float32),
        ],
    )(x, W1, W2)
    numel = B * C * T
    total_loss = loss[0, 0] * (1.25 / (2.0 * numel))
    return (qout, total_loss, idx)
